# baseline (speedup 1.0000x reference)
import functools

import jax
import jax.numpy as jnp
from jax import lax
from jax.experimental import pallas as pl
from jax.experimental.pallas import tpu as pltpu
from jax.experimental.pallas import tpu_sc as plsc

N_NODES = 10000
N_EDGES = 320000
D = 128

NC = 2
NS = 16
NW = NC * NS
CHUNK = 80
STRIPE_CHUNKS = (55, 70)
STRIPE_PHASES = ((5, 11), (2, 35))
STRIPE_EPW = tuple(c * CHUNK for c in STRIPE_CHUNKS)
STRIPE_E = tuple(e * NW for e in STRIPE_EPW)
STRIPE_BASE = (0, STRIPE_E[0], STRIPE_E[0] + STRIPE_E[1])
MLP_BLK = 2560
N_PAD = 10240
ROWS_PER_SUB = N_PAD // NS


def _mlp_block_kernel(x_ref, w_ref, b_ref, o_ref):
    h = jnp.dot(x_ref[...], w_ref[...], preferred_element_type=jnp.float32)
    h = jnp.maximum(h + b_ref[...], 0.0)
    o_ref[...] = jax.nn.sigmoid(h)


def _edge_mlp(edge_attr, W1, b1, start_row, n_rows):
    start_blk = start_row // MLP_BLK
    grid = n_rows // MLP_BLK
    return pl.pallas_call(
        _mlp_block_kernel,
        grid=(grid,),
        in_specs=[
            pl.BlockSpec((MLP_BLK, D), lambda i: (start_blk + i, 0)),
            pl.BlockSpec((D, D), lambda i: (0, 0)),
            pl.BlockSpec((1, D), lambda i: (0, 0)),
        ],
        out_specs=pl.BlockSpec((MLP_BLK, D), lambda i: (i, 0)),
        out_shape=jax.ShapeDtypeStruct((n_rows, D), jnp.float32),
    )(edge_attr, W1, b1.reshape(1, D))


def _sum_mlp_kernel(p_ref, w_ref, b_ref, o_ref):
    s = p_ref[0] + p_ref[1]
    h = jnp.dot(s, w_ref[...], preferred_element_type=jnp.float32)
    h = jnp.maximum(h + b_ref[...], 0.0)
    o_ref[...] = jax.nn.sigmoid(h)


def _node_mlp(partials, W2, b2):
    blk = 2000
    grid = N_NODES // blk
    return pl.pallas_call(
        _sum_mlp_kernel,
        grid=(grid,),
        in_specs=[
            pl.BlockSpec((NC, blk, D), lambda i: (0, i, 0)),
            pl.BlockSpec((D, D), lambda i: (0, 0)),
            pl.BlockSpec((1, D), lambda i: (0, 0)),
        ],
        out_specs=pl.BlockSpec((blk, D), lambda i: (i, 0)),
        out_shape=jax.ShapeDtypeStruct((N_NODES, D), jnp.float32),
    )(partials, W2, b2.reshape(1, D))


def _make_sc_body(e_per_w, n_phase, pch, chained):
    n_chunks = n_phase * pch
    assert pch % 2 == 1

    def body(h_hbm, col_hbm, init_hbm, out_hbm,
             idx0, idx1, buf0, buf1, acc_shared,
             gsem0, gsem1, ssem0, ssem1):
        cid = lax.axis_index("c")
        sid = lax.axis_index("s")
        wid = cid * NS + sid

        if chained:
            init_src = init_hbm.at[cid].at[
                pl.ds(sid * ROWS_PER_SUB, ROWS_PER_SUB)]
        else:
            init_src = init_hbm
        pltpu.sync_copy(
            init_src,
            acc_shared.at[pl.ds(sid * ROWS_PER_SUB, ROWS_PER_SUB)],
        )
        plsc.subcore_barrier()

        base = wid * e_per_w
        bufs = (buf0, buf1)
        gsems = (gsem0, gsem1)
        ssems = (ssem0, ssem1)
        idxbufs = (idx0, idx1)

        def fire_gather(c, m):
            pltpu.async_copy(
                h_hbm.at[pl.ds(base + c * CHUNK, CHUNK)], bufs[m], gsems[m])

        def wait_gather(m):
            pltpu.make_async_copy(
                h_hbm.at[pl.ds(0, CHUNK)], bufs[m], gsems[m]).wait()

        def wait_scatter(m):
            pltpu.make_async_copy(
                bufs[m], acc_shared.at[pl.ds(0, CHUNK)], ssems[m]).wait()

        def do_chunk(c, m, idxrow, first=False, last=False):
            wait_gather(m)
            pltpu.async_copy(bufs[m], acc_shared.at[idxrow], ssems[m], add=True)
            if not first:
                wait_scatter(1 - m)
            if not last:
                fire_gather(c + 1, 1 - m)

        pltpu.sync_copy(col_hbm.at[wid, 0], idx0)
        fire_gather(0, 0)
        do_chunk(0, 0, idx0.at[0], first=True)

        for p in range(n_phase):
            ib = idxbufs[p % 2]
            if p > 0:
                pltpu.sync_copy(col_hbm.at[wid, p], ib)

            if p == 0:
                def inner0(o, carry):
                    for b in range(2):
                        j = 1 + 2 * o + b
                        do_chunk(j, (1 + b) % 2, idx0.at[j])
                    return carry
                lax.fori_loop(0, pch // 2, inner0, 0)
            else:
                def inner(o, carry, p=p, ib=ib):
                    for b in range(2):
                        j = 2 * o + b
                        do_chunk(p * pch + j, (p + b) % 2, ib.at[j])
                    return carry
                lax.fori_loop(0, pch // 2, inner, 0)
                j = pch - 1
                c = p * pch + j
                do_chunk(c, (p + j) % 2, ib.at[j], last=(c == n_chunks - 1))

        wait_scatter((n_chunks - 1) % 2)
        plsc.subcore_barrier()

        pltpu.sync_copy(
            acc_shared.at[pl.ds(sid * ROWS_PER_SUB, ROWS_PER_SUB)],
            out_hbm.at[cid].at[pl.ds(sid * ROWS_PER_SUB, ROWS_PER_SUB)],
        )

    return body


def _sc_scatter(h, col4, init, e_per_w, n_phase, pch, chained):
    mesh = plsc.VectorSubcoreMesh(
        core_axis_name="c", subcore_axis_name="s", num_cores=NC, num_subcores=NS
    )
    f = pl.kernel(
        _make_sc_body(e_per_w, n_phase, pch, chained),
        out_type=jax.ShapeDtypeStruct((NC, N_PAD, D), jnp.float32),
        mesh=mesh,
        scratch_types=[
            pltpu.VMEM((pch, CHUNK), jnp.int32),
            pltpu.VMEM((pch, CHUNK), jnp.int32),
            pltpu.VMEM((CHUNK, D), jnp.float32),
            pltpu.VMEM((CHUNK, D), jnp.float32),
            pltpu.VMEM_SHARED((N_PAD, D), jnp.float32),
            pltpu.SemaphoreType.DMA,
            pltpu.SemaphoreType.DMA,
            pltpu.SemaphoreType.DMA,
            pltpu.SemaphoreType.DMA,
        ],
    )
    return f(h, col4, init)


@jax.jit
def kernel(x, edge_index, edge_attr, u, batch, W1, b1, W2, b2):
    col = edge_index[1].astype(jnp.int32)
    acc = jnp.zeros((ROWS_PER_SUB, D), jnp.float32)
    for s in range(len(STRIPE_CHUNKS)):
        n_phase, pch = STRIPE_PHASES[s]
        h = _edge_mlp(edge_attr, W1, b1, STRIPE_BASE[s], STRIPE_E[s])
        cols = lax.slice_in_dim(col, STRIPE_BASE[s],
                                STRIPE_BASE[s] + STRIPE_E[s], axis=0)
        col4 = cols.reshape(NW, n_phase, pch, CHUNK)
        acc = _sc_scatter(h, col4, acc, STRIPE_EPW[s], n_phase, pch,
                          chained=(s > 0))
    return _node_mlp(acc, W2, b2)


# speedup vs baseline: 1.0108x; 1.0108x over previous
import jax
import jax.numpy as jnp
from jax import lax
from jax.experimental import pallas as pl
from jax.experimental.pallas import tpu as pltpu
from jax.experimental.pallas import tpu_sc as plsc

N_NODES = 10000
N_EDGES = 320000
D = 128

NC = 2
NS = 16
NW = NC * NS
CHUNK = 80
STRIPE_CHUNKS = (63, 62)
STRIPE_PHASES = ((3, 21), (2, 31))
STRIPE_EPW = tuple(c * CHUNK for c in STRIPE_CHUNKS)
STRIPE_E = tuple(e * NW for e in STRIPE_EPW)
STRIPE_BASE = (0, STRIPE_E[0])
MLP_BLK = 2560
N_PAD = 10240
ROWS_PER_SUB = N_PAD // NS


def _mlp_block_kernel(x_ref, w_ref, b_ref, o_ref):
    h = jnp.dot(x_ref[...], w_ref[...], preferred_element_type=jnp.float32)
    h = jnp.maximum(h + b_ref[...], 0.0)
    o_ref[...] = jax.nn.sigmoid(h)


def _edge_mlp(edge_attr, W1, b1, start_row, n_rows):
    start_blk = start_row // MLP_BLK
    grid = n_rows // MLP_BLK
    return pl.pallas_call(
        _mlp_block_kernel,
        grid=(grid,),
        in_specs=[
            pl.BlockSpec((MLP_BLK, D), lambda i: (start_blk + i, 0)),
            pl.BlockSpec((D, D), lambda i: (0, 0)),
            pl.BlockSpec((1, D), lambda i: (0, 0)),
        ],
        out_specs=pl.BlockSpec((MLP_BLK, D), lambda i: (i, 0)),
        out_shape=jax.ShapeDtypeStruct((n_rows, D), jnp.float32),
    )(edge_attr, W1, b1.reshape(1, D))


def _sum_mlp_kernel(p_ref, w_ref, b_ref, o_ref):
    s = p_ref[0] + p_ref[1]
    h = jnp.dot(s, w_ref[...], preferred_element_type=jnp.float32)
    h = jnp.maximum(h + b_ref[...], 0.0)
    o_ref[...] = jax.nn.sigmoid(h)


def _node_mlp(partials, W2, b2):
    blk = 2000
    grid = N_NODES // blk
    return pl.pallas_call(
        _sum_mlp_kernel,
        grid=(grid,),
        in_specs=[
            pl.BlockSpec((NC, blk, D), lambda i: (0, i, 0)),
            pl.BlockSpec((D, D), lambda i: (0, 0)),
            pl.BlockSpec((1, D), lambda i: (0, 0)),
        ],
        out_specs=pl.BlockSpec((blk, D), lambda i: (i, 0)),
        out_shape=jax.ShapeDtypeStruct((N_NODES, D), jnp.float32),
    )(partials, W2, b2.reshape(1, D))


def _make_sc_body(e_per_w, n_phase, pch, chained):
    n_chunks = n_phase * pch
    assert pch % 2 == 1

    def body(h_hbm, col_hbm, init_hbm, out_hbm,
             idx0, idx1, buf0, buf1, acc_shared,
             gsem0, gsem1, ssem0, ssem1):
        cid = lax.axis_index("c")
        sid = lax.axis_index("s")
        wid = cid * NS + sid

        if chained:
            init_src = init_hbm.at[cid].at[
                pl.ds(sid * ROWS_PER_SUB, ROWS_PER_SUB)]
        else:
            init_src = init_hbm
        pltpu.sync_copy(
            init_src,
            acc_shared.at[pl.ds(sid * ROWS_PER_SUB, ROWS_PER_SUB)],
        )
        plsc.subcore_barrier()

        base = wid * e_per_w
        bufs = (buf0, buf1)
        gsems = (gsem0, gsem1)
        ssems = (ssem0, ssem1)
        idxbufs = (idx0, idx1)

        def fire_gather(c, m):
            pltpu.async_copy(
                h_hbm.at[pl.ds(base + c * CHUNK, CHUNK)], bufs[m], gsems[m])

        def wait_gather(m):
            pltpu.make_async_copy(
                h_hbm.at[pl.ds(0, CHUNK)], bufs[m], gsems[m]).wait()

        def wait_scatter(m):
            pltpu.make_async_copy(
                bufs[m], acc_shared.at[pl.ds(0, CHUNK)], ssems[m]).wait()

        def do_chunk(c, m, idxrow, first=False, last=False):
            wait_gather(m)
            pltpu.async_copy(bufs[m], acc_shared.at[idxrow], ssems[m], add=True)
            if not first:
                wait_scatter(1 - m)
            if not last:
                fire_gather(c + 1, 1 - m)

        pltpu.sync_copy(col_hbm.at[wid, 0], idx0)
        fire_gather(0, 0)
        do_chunk(0, 0, idx0.at[0], first=True)

        for p in range(n_phase):
            ib = idxbufs[p % 2]
            if p > 0:
                pltpu.sync_copy(col_hbm.at[wid, p], ib)

            if p == 0:
                def inner0(o, carry):
                    for b in range(2):
                        j = 1 + 2 * o + b
                        do_chunk(j, (1 + b) % 2, idx0.at[j])
                    return carry
                lax.fori_loop(0, pch // 2, inner0, 0)
            else:
                def inner(o, carry, p=p, ib=ib):
                    for b in range(2):
                        j = 2 * o + b
                        do_chunk(p * pch + j, (p + b) % 2, ib.at[j])
                    return carry
                lax.fori_loop(0, pch // 2, inner, 0)
                j = pch - 1
                c = p * pch + j
                do_chunk(c, (p + j) % 2, ib.at[j], last=(c == n_chunks - 1))

        wait_scatter((n_chunks - 1) % 2)
        plsc.subcore_barrier()

        pltpu.sync_copy(
            acc_shared.at[pl.ds(sid * ROWS_PER_SUB, ROWS_PER_SUB)],
            out_hbm.at[cid].at[pl.ds(sid * ROWS_PER_SUB, ROWS_PER_SUB)],
        )

    return body


def _sc_scatter(h, col4, init, e_per_w, n_phase, pch, chained):
    mesh = plsc.VectorSubcoreMesh(
        core_axis_name="c", subcore_axis_name="s", num_cores=NC, num_subcores=NS
    )
    f = pl.kernel(
        _make_sc_body(e_per_w, n_phase, pch, chained),
        out_type=jax.ShapeDtypeStruct((NC, N_PAD, D), jnp.float32),
        mesh=mesh,
        scratch_types=[
            pltpu.VMEM((pch, CHUNK), jnp.int32),
            pltpu.VMEM((pch, CHUNK), jnp.int32),
            pltpu.VMEM((CHUNK, D), jnp.float32),
            pltpu.VMEM((CHUNK, D), jnp.float32),
            pltpu.VMEM_SHARED((N_PAD, D), jnp.float32),
            pltpu.SemaphoreType.DMA,
            pltpu.SemaphoreType.DMA,
            pltpu.SemaphoreType.DMA,
            pltpu.SemaphoreType.DMA,
        ],
    )
    return f(h, col4, init)


@jax.jit
def kernel(x, edge_index, edge_attr, u, batch, W1, b1, W2, b2):
    col = edge_index[1].astype(jnp.int32)
    acc = jnp.zeros((ROWS_PER_SUB, D), jnp.float32)
    for s in range(len(STRIPE_CHUNKS)):
        n_phase, pch = STRIPE_PHASES[s]
        h = _edge_mlp(edge_attr, W1, b1, STRIPE_BASE[s], STRIPE_E[s])
        cols = lax.slice_in_dim(col, STRIPE_BASE[s],
                                STRIPE_BASE[s] + STRIPE_E[s], axis=0)
        col4 = cols.reshape(NW, n_phase, pch, CHUNK)
        acc = _sc_scatter(h, col4, acc, STRIPE_EPW[s], n_phase, pch,
                          chained=(s > 0))
    return _node_mlp(acc, W2, b2)
